# Initial kernel scaffold; baseline (speedup 1.0000x reference)
import jax
import jax.numpy as jnp
from jax.experimental import pallas as pl


def kernel(coords, bonds, encoded, t, answer, W0, B0, W1, B1):
    raise NotImplementedError("write your pallas kernel here")


# speedup vs baseline: 3.0101x; 3.0101x over previous
import functools

import jax
import jax.numpy as jnp
from jax import lax
from jax.experimental import pallas as pl
from jax.experimental.pallas import tpu as pltpu
from jax.experimental.pallas import tpu_sc as plsc

N = 10000
E = 320000
D = 128
NC, NS = 2, 16
NW = NC * NS
EPW = E // NW
C = 80
NCHUNK = EPW // C
GROUPS = C // 16
N3 = 3 * N
N3P = 30720


def _proj_body(x_ref, a0_ref, a1_ref, ch_ref, g0_ref, g1_ref):
    x = x_ref[...]
    ch = ch_ref[0:1, :]
    g0_ref[...] = jnp.dot(x, a0_ref[...], preferred_element_type=jnp.float32,
                        precision=lax.Precision.HIGHEST) + ch
    g1_ref[...] = jnp.dot(x, a1_ref[...], preferred_element_type=jnp.float32,
                        precision=lax.Precision.HIGHEST) + ch


_PBLK = 1000
_proj = pl.pallas_call(
    _proj_body,
    grid=(N // _PBLK,),
    in_specs=[
        pl.BlockSpec((_PBLK, D), lambda i: (i, 0)),
        pl.BlockSpec((D, D), lambda i: (0, 0)),
        pl.BlockSpec((D, D), lambda i: (0, 0)),
        pl.BlockSpec((8, D), lambda i: (0, 0)),
    ],
    out_specs=[
        pl.BlockSpec((_PBLK, D), lambda i: (i, 0)),
        pl.BlockSpec((_PBLK, D), lambda i: (i, 0)),
    ],
    out_shape=[
        jax.ShapeDtypeStruct((N, D), jnp.float32),
        jax.ShapeDtypeStruct((N, D), jnp.float32),
    ],
)


_mesh = plsc.VectorSubcoreMesh(
    core_axis_name="c", subcore_axis_name="s", num_cores=NC, num_subcores=NS)


@functools.partial(
    pl.kernel,
    out_type=jax.ShapeDtypeStruct((NW, N3P), jnp.float32),
    mesh=_mesh,
    scratch_types=[
        pltpu.VMEM((C,), jnp.int32),
        pltpu.VMEM((C,), jnp.int32),
        pltpu.VMEM((C, D), jnp.float32),
        pltpu.VMEM((C, D), jnp.float32),
        pltpu.VMEM((N,), jnp.float32),
        pltpu.VMEM((N,), jnp.float32),
        pltpu.VMEM((N,), jnp.float32),
        pltpu.VMEM((8, D), jnp.float32),
        pltpu.SMEM((8, D), jnp.float32),
        pltpu.VMEM((N3P,), jnp.float32),
        pltpu.SemaphoreType.DMA,
        pltpu.SemaphoreType.DMA,
    ],
    compiler_params=pltpu.CompilerParams(needs_layout_passes=False),
)
def _sc_edges(g0_hbm, g1_hbm, i0_hbm, i1_hbm, cx_hbm, cy_hbm, cz_hbm, misc_hbm,
              out_hbm, i0v, i1v, r0, r1, cxv, cyv, czv, miscb, miscv, acc,
              sem0, sem1):
    wid = lax.axis_index("s") * NC + lax.axis_index("c")
    base_w = wid * EPW

    pltpu.sync_copy(cx_hbm, cxv)
    pltpu.sync_copy(cy_hbm, cyv)
    pltpu.sync_copy(cz_hbm, czv)
    pltpu.sync_copy(misc_hbm, miscb)
    for i in range(3):
        for jc in range(D // 16):
            vv = miscb[i, pl.ds(jc * 16, 16)]
            for k2 in range(16):
                miscv[i, jc * 16 + k2] = vv[k2]
    vb = miscb[3, pl.ds(0, 16)]
    miscv[3, 0] = vb[0]
    miscv[3, 1] = vb[1]

    zer16 = jnp.zeros((16,), jnp.float32)

    def _zero(i, carry):
        acc[pl.ds(i * 16, 16)] = zer16
        return carry
    lax.fori_loop(0, N3P // 16, _zero, 0)

    lanes = lax.iota(jnp.int32, 16)

    def _chunk(k, carry):
        base = base_w + k * C
        pltpu.sync_copy(i0_hbm.at[pl.ds(base, C)], i0v)
        pltpu.sync_copy(i1_hbm.at[pl.ds(base, C)], i1v)
        cp0 = pltpu.async_copy(g0_hbm.at[i0v], r0, sem0)
        cp1 = pltpu.async_copy(g1_hbm.at[i1v], r1, sem1)
        cp0.wait()
        cp1.wait()
        for g in range(GROUPS):
            e0 = g * 16
            i0g = i0v[pl.ds(e0, 16)]
            i1g = i1v[pl.ds(e0, 16)]
            x0 = plsc.load_gather(cxv, [i0g])
            y0 = plsc.load_gather(cyv, [i0g])
            z0 = plsc.load_gather(czv, [i0g])
            x1 = plsc.load_gather(cxv, [i1g])
            y1 = plsc.load_gather(cyv, [i1g])
            z1 = plsc.load_gather(czv, [i1g])
            drx = x0 - x1
            dry = y0 - y1
            drz = z0 - z1
            dl2 = jnp.maximum(drx * drx + dry * dry + drz * drz,
                              jnp.float32(1e-12))
            xi = plsc.bitcast(dl2, jnp.int32)
            xi = jnp.int32(0x5F3759DF) - (xi >> 1)
            ys = plsc.bitcast(xi, jnp.float32)
            for _ in range(3):
                ys = ys * (jnp.float32(1.5)
                           - jnp.float32(0.5) * dl2 * ys * ys)
            rs = ys
            dl = dl2 * rs

            ev = lanes + jnp.int32(e0)

            def _dbody(d, uv):
                u, v = uv
                dsp = jnp.full((16,), d, jnp.int32)
                a = plsc.load_gather(r0, [ev, dsp])
                b = plsc.load_gather(r1, [ev, dsp])
                h = a + b + dl * miscv[0, d]
                act = jnp.maximum(h, jnp.float32(0.001) * h)
                u = u + act * miscv[1, d]
                v = v + act * miscv[2, d]
                return (u, v)

            u, v = lax.fori_loop(0, D, _dbody, (zer16, zer16))
            d0 = u + miscv[3, 0]
            d1 = v + miscv[3, 1]
            dhx = drx * rs
            dhy = dry * rs
            dhz = drz * rs
            b0 = i0g * 3
            plsc.addupdate_scatter(acc, [b0], d0 * dhx)
            plsc.addupdate_scatter(acc, [b0 + 1], d0 * dhy)
            plsc.addupdate_scatter(acc, [b0 + 2], d0 * dhz)
            b1 = i1g * 3
            plsc.addupdate_scatter(acc, [b1], d1 * dhx)
            plsc.addupdate_scatter(acc, [b1 + 1], d1 * dhy)
            plsc.addupdate_scatter(acc, [b1 + 2], d1 * dhz)
        return carry

    lax.fori_loop(0, NCHUNK, _chunk, 0)
    pltpu.sync_copy(acc, out_hbm.at[wid])


def _red_body(p_ref, a_ref, o_ref):
    o_ref[...] = a_ref[...] + jnp.sum(p_ref[...], axis=0)


_reduce = pl.pallas_call(
    _red_body,
    out_shape=jax.ShapeDtypeStruct((N3P,), jnp.float32),
)


def kernel(coords, bonds, encoded, t, answer, W0, B0, W1, B1):
    a0t = W0[:, :D].T
    a1t = W0[:, D:2 * D].T
    ch = 0.5 * (t[0] * W0[:, 2 * D] + B0)
    ch8 = jnp.zeros((8, D), jnp.float32).at[0].set(ch)
    misc = jnp.zeros((8, D), jnp.float32)
    misc = misc.at[0].set(W0[:, 2 * D + 1])
    misc = misc.at[1].set(-0.5 * W1[0])
    misc = misc.at[2].set(0.5 * W1[1])
    misc = misc.at[3, 0].set(-0.5 * B1[0])
    misc = misc.at[3, 1].set(0.5 * B1[1])

    g0, g1 = _proj(encoded, a0t, a1t, ch8)

    i0 = bonds[:, 0]
    i1 = bonds[:, 1]
    cx = coords[:, 0, 0]
    cy = coords[:, 0, 1]
    cz = coords[:, 0, 2]

    partials = _sc_edges(g0, g1, i0, i1, cx, cy, cz, misc)

    ans_pad = jnp.pad(answer.reshape(-1), (0, N3P - N3))
    out = _reduce(partials, ans_pad)
    return out[:N3].reshape(N, 1, 3)


# speedup vs baseline: 3.0909x; 1.0268x over previous
import functools

import jax
import jax.numpy as jnp
from jax import lax
from jax.experimental import pallas as pl
from jax.experimental.pallas import tpu as pltpu
from jax.experimental.pallas import tpu_sc as plsc

N = 10000
E = 320000
D = 128
NC, NS = 2, 16
NW = NC * NS
EPW = E // NW
C = 80
NCHUNK = EPW // C
GROUPS = C // 16
N3 = 3 * N
N3P = 30720


def _proj_body(x_ref, a0_ref, a1_ref, ch_ref, g0_ref, g1_ref):
    x = x_ref[...]
    ch = ch_ref[0:1, :]
    g0_ref[...] = jnp.dot(x, a0_ref[...], preferred_element_type=jnp.float32,
                        precision=lax.Precision.HIGHEST) + ch
    g1_ref[...] = jnp.dot(x, a1_ref[...], preferred_element_type=jnp.float32,
                        precision=lax.Precision.HIGHEST) + ch


_PBLK = 1000
_proj = pl.pallas_call(
    _proj_body,
    grid=(N // _PBLK,),
    in_specs=[
        pl.BlockSpec((_PBLK, D), lambda i: (i, 0)),
        pl.BlockSpec((D, D), lambda i: (0, 0)),
        pl.BlockSpec((D, D), lambda i: (0, 0)),
        pl.BlockSpec((8, D), lambda i: (0, 0)),
    ],
    out_specs=[
        pl.BlockSpec((_PBLK, D), lambda i: (i, 0)),
        pl.BlockSpec((_PBLK, D), lambda i: (i, 0)),
    ],
    out_shape=[
        jax.ShapeDtypeStruct((N, D), jnp.float32),
        jax.ShapeDtypeStruct((N, D), jnp.float32),
    ],
)


_mesh = plsc.VectorSubcoreMesh(
    core_axis_name="c", subcore_axis_name="s", num_cores=NC, num_subcores=NS)


@functools.partial(
    pl.kernel,
    out_type=jax.ShapeDtypeStruct((NW, N3P), jnp.float32),
    mesh=_mesh,
    scratch_types=[
        pltpu.VMEM((C,), jnp.int32),
        pltpu.VMEM((C,), jnp.int32),
        pltpu.VMEM((C, D), jnp.float32),
        pltpu.VMEM((C, D), jnp.float32),
        pltpu.VMEM((N,), jnp.float32),
        pltpu.VMEM((N,), jnp.float32),
        pltpu.VMEM((N,), jnp.float32),
        pltpu.VMEM((8, D), jnp.float32),
        pltpu.SMEM((8, D), jnp.float32),
        pltpu.VMEM((N3P,), jnp.float32),
        pltpu.SemaphoreType.DMA,
        pltpu.SemaphoreType.DMA,
    ],
    compiler_params=pltpu.CompilerParams(needs_layout_passes=False),
)
def _sc_edges(g0_hbm, g1_hbm, i0_hbm, i1_hbm, cx_hbm, cy_hbm, cz_hbm, misc_hbm,
              out_hbm, i0v, i1v, r0, r1, cxv, cyv, czv, miscb, miscv, acc,
              sem0, sem1):
    wid = lax.axis_index("s") * NC + lax.axis_index("c")
    base_w = wid * EPW

    pltpu.sync_copy(cx_hbm, cxv)
    pltpu.sync_copy(cy_hbm, cyv)
    pltpu.sync_copy(cz_hbm, czv)
    pltpu.sync_copy(misc_hbm, miscb)
    for i in range(3):
        for jc in range(D // 16):
            vv = miscb[i, pl.ds(jc * 16, 16)]
            for k2 in range(16):
                miscv[i, jc * 16 + k2] = vv[k2]
    vb = miscb[3, pl.ds(0, 16)]
    miscv[3, 0] = vb[0]
    miscv[3, 1] = vb[1]

    zer16 = jnp.zeros((16,), jnp.float32)

    def _zero(i, carry):
        acc[pl.ds(i * 16, 16)] = zer16
        return carry
    lax.fori_loop(0, N3P // 16, _zero, 0)

    lanes = lax.iota(jnp.int32, 16)

    def _chunk(k, carry):
        base = base_w + k * C
        pltpu.sync_copy(i0_hbm.at[pl.ds(base, C)], i0v)
        pltpu.sync_copy(i1_hbm.at[pl.ds(base, C)], i1v)
        cp0 = pltpu.async_copy(g0_hbm.at[i0v], r0, sem0)
        cp1 = pltpu.async_copy(g1_hbm.at[i1v], r1, sem1)
        cp0.wait()
        cp1.wait()
        for g in range(GROUPS):
            e0 = g * 16
            i0g = i0v[pl.ds(e0, 16)]
            i1g = i1v[pl.ds(e0, 16)]
            x0 = plsc.load_gather(cxv, [i0g])
            y0 = plsc.load_gather(cyv, [i0g])
            z0 = plsc.load_gather(czv, [i0g])
            x1 = plsc.load_gather(cxv, [i1g])
            y1 = plsc.load_gather(cyv, [i1g])
            z1 = plsc.load_gather(czv, [i1g])
            drx = x0 - x1
            dry = y0 - y1
            drz = z0 - z1
            dl2 = jnp.maximum(drx * drx + dry * dry + drz * drz,
                              jnp.float32(1e-12))
            xi = plsc.bitcast(dl2, jnp.int32)
            xi = jnp.int32(0x5F3759DF) - (xi >> 1)
            ys = plsc.bitcast(xi, jnp.float32)
            for _ in range(3):
                ys = ys * (jnp.float32(1.5)
                           - jnp.float32(0.5) * dl2 * ys * ys)
            rs = ys
            dl = dl2 * rs

            ev = lanes + jnp.int32(e0)
            dsp0 = jnp.zeros((16,), jnp.int32)

            def _dbody(i, carry):
                u, v, idx = carry
                for dj in range(8):
                    d = i * 8 + dj
                    a = plsc.load_gather(r0, [ev, idx])
                    b = plsc.load_gather(r1, [ev, idx])
                    h = a + b + dl * miscv[0, d]
                    act = jnp.maximum(h, jnp.float32(0.001) * h)
                    u = u + act * miscv[1, d]
                    v = v + act * miscv[2, d]
                    idx = idx + 1
                return (u, v, idx)

            u, v, _ = lax.fori_loop(0, D // 8, _dbody, (zer16, zer16, dsp0))
            d0 = u + miscv[3, 0]
            d1 = v + miscv[3, 1]
            dhx = drx * rs
            dhy = dry * rs
            dhz = drz * rs
            b0 = i0g * 3
            plsc.addupdate_scatter(acc, [b0], d0 * dhx)
            plsc.addupdate_scatter(acc, [b0 + 1], d0 * dhy)
            plsc.addupdate_scatter(acc, [b0 + 2], d0 * dhz)
            b1 = i1g * 3
            plsc.addupdate_scatter(acc, [b1], d1 * dhx)
            plsc.addupdate_scatter(acc, [b1 + 1], d1 * dhy)
            plsc.addupdate_scatter(acc, [b1 + 2], d1 * dhz)
        return carry

    lax.fori_loop(0, NCHUNK, _chunk, 0)
    pltpu.sync_copy(acc, out_hbm.at[wid])


def _red_body(p_ref, a_ref, o_ref):
    o_ref[...] = a_ref[...] + jnp.sum(p_ref[...], axis=0)


_reduce = pl.pallas_call(
    _red_body,
    out_shape=jax.ShapeDtypeStruct((N3P,), jnp.float32),
)


def kernel(coords, bonds, encoded, t, answer, W0, B0, W1, B1):
    a0t = W0[:, :D].T
    a1t = W0[:, D:2 * D].T
    ch = 0.5 * (t[0] * W0[:, 2 * D] + B0)
    ch8 = jnp.zeros((8, D), jnp.float32).at[0].set(ch)
    misc = jnp.zeros((8, D), jnp.float32)
    misc = misc.at[0].set(W0[:, 2 * D + 1])
    misc = misc.at[1].set(-0.5 * W1[0])
    misc = misc.at[2].set(0.5 * W1[1])
    misc = misc.at[3, 0].set(-0.5 * B1[0])
    misc = misc.at[3, 1].set(0.5 * B1[1])

    g0, g1 = _proj(encoded, a0t, a1t, ch8)

    i0 = bonds[:, 0]
    i1 = bonds[:, 1]
    cx = coords[:, 0, 0]
    cy = coords[:, 0, 1]
    cz = coords[:, 0, 2]

    partials = _sc_edges(g0, g1, i0, i1, cx, cy, cz, misc)

    ans_pad = jnp.pad(answer.reshape(-1), (0, N3P - N3))
    out = _reduce(partials, ans_pad)
    return out[:N3].reshape(N, 1, 3)


# speedup vs baseline: 3.5815x; 1.1587x over previous
import functools

import jax
import jax.numpy as jnp
from jax import lax
from jax.experimental import pallas as pl
from jax.experimental.pallas import tpu as pltpu
from jax.experimental.pallas import tpu_sc as plsc

N = 10000
E = 320000
D = 128
NC, NS = 2, 16
NW = NC * NS
EPW = E // NW
C = 80
NCHUNK = EPW // C
GROUPS = C // 16
N3 = 3 * N
N3P = 30720


def _proj_body(x_ref, a0_ref, a1_ref, ch_ref, g0_ref, g1_ref):
    x = x_ref[...]
    ch = ch_ref[0:1, :]
    g0_ref[...] = jnp.dot(x, a0_ref[...], preferred_element_type=jnp.float32,
                        precision=lax.Precision.HIGHEST) + ch
    g1_ref[...] = jnp.dot(x, a1_ref[...], preferred_element_type=jnp.float32,
                        precision=lax.Precision.HIGHEST) + ch


_PBLK = 1000
_proj = pl.pallas_call(
    _proj_body,
    grid=(N // _PBLK,),
    in_specs=[
        pl.BlockSpec((_PBLK, D), lambda i: (i, 0)),
        pl.BlockSpec((D, D), lambda i: (0, 0)),
        pl.BlockSpec((D, D), lambda i: (0, 0)),
        pl.BlockSpec((8, D), lambda i: (0, 0)),
    ],
    out_specs=[
        pl.BlockSpec((_PBLK, D), lambda i: (i, 0)),
        pl.BlockSpec((_PBLK, D), lambda i: (i, 0)),
    ],
    out_shape=[
        jax.ShapeDtypeStruct((N, D), jnp.float32),
        jax.ShapeDtypeStruct((N, D), jnp.float32),
    ],
)


_mesh = plsc.VectorSubcoreMesh(
    core_axis_name="c", subcore_axis_name="s", num_cores=NC, num_subcores=NS)


@functools.partial(
    pl.kernel,
    out_type=jax.ShapeDtypeStruct((NW, N3P), jnp.float32),
    mesh=_mesh,
    scratch_types=[
        pltpu.VMEM((EPW,), jnp.int32),
        pltpu.VMEM((EPW,), jnp.int32),
        pltpu.VMEM((2, C, D), jnp.float32),
        pltpu.VMEM((2, C, D), jnp.float32),
        pltpu.VMEM((N,), jnp.float32),
        pltpu.VMEM((N,), jnp.float32),
        pltpu.VMEM((N,), jnp.float32),
        pltpu.VMEM((8, D), jnp.float32),
        pltpu.SMEM((8, D), jnp.float32),
        pltpu.VMEM((N3P,), jnp.float32),
        pltpu.SemaphoreType.DMA,
        pltpu.SemaphoreType.DMA,
        pltpu.SemaphoreType.DMA,
        pltpu.SemaphoreType.DMA,
    ],
    compiler_params=pltpu.CompilerParams(needs_layout_passes=False),
)
def _sc_edges(g0_hbm, g1_hbm, i0_hbm, i1_hbm, cx_hbm, cy_hbm, cz_hbm, misc_hbm,
              out_hbm, i0all, i1all, r0, r1, cxv, cyv, czv, miscb, miscv, acc,
              *sems):
    wid = lax.axis_index("s") * NC + lax.axis_index("c")
    base_w = wid * EPW

    pltpu.sync_copy(i0_hbm.at[pl.ds(base_w, EPW)], i0all)
    pltpu.sync_copy(i1_hbm.at[pl.ds(base_w, EPW)], i1all)
    pltpu.sync_copy(cx_hbm, cxv)
    pltpu.sync_copy(cy_hbm, cyv)
    pltpu.sync_copy(cz_hbm, czv)
    pltpu.sync_copy(misc_hbm, miscb)
    for i in range(3):
        for jc in range(D // 16):
            vv = miscb[i, pl.ds(jc * 16, 16)]
            for k2 in range(16):
                miscv[i, jc * 16 + k2] = vv[k2]
    vb = miscb[3, pl.ds(0, 16)]
    miscv[3, 0] = vb[0]
    miscv[3, 1] = vb[1]

    zer16 = jnp.zeros((16,), jnp.float32)

    def _zero(i, carry):
        acc[pl.ds(i * 16, 16)] = zer16
        return carry
    lax.fori_loop(0, N3P // 16, _zero, 0)

    lanes = lax.iota(jnp.int32, 16)

    def _issue(k, b):
        off = k * C
        pltpu.async_copy(g0_hbm.at[i0all.at[pl.ds(off, C)]], r0.at[b], sems[b])
        pltpu.async_copy(g1_hbm.at[i1all.at[pl.ds(off, C)]], r1.at[b],
                         sems[2 + b])

    def _waitbuf(k, b):
        off = k * C
        pltpu.make_async_copy(
            g0_hbm.at[i0all.at[pl.ds(off, C)]], r0.at[b], sems[b]).wait()
        pltpu.make_async_copy(
            g1_hbm.at[i1all.at[pl.ds(off, C)]], r1.at[b], sems[2 + b]).wait()

    def _compute(k, b):
        r0b = r0.at[b]
        r1b = r1.at[b]
        for g in range(GROUPS):
            e0 = g * 16
            eoff = k * C + e0
            i0g = i0all[pl.ds(eoff, 16)]
            i1g = i1all[pl.ds(eoff, 16)]
            x0 = plsc.load_gather(cxv, [i0g])
            y0 = plsc.load_gather(cyv, [i0g])
            z0 = plsc.load_gather(czv, [i0g])
            x1 = plsc.load_gather(cxv, [i1g])
            y1 = plsc.load_gather(cyv, [i1g])
            z1 = plsc.load_gather(czv, [i1g])
            drx = x0 - x1
            dry = y0 - y1
            drz = z0 - z1
            dl2 = jnp.maximum(drx * drx + dry * dry + drz * drz,
                              jnp.float32(1e-12))
            xi = plsc.bitcast(dl2, jnp.int32)
            xi = jnp.int32(0x5F3759DF) - (xi >> 1)
            ys = plsc.bitcast(xi, jnp.float32)
            for _ in range(3):
                ys = ys * (jnp.float32(1.5)
                           - jnp.float32(0.5) * dl2 * ys * ys)
            rs = ys
            dl = dl2 * rs

            ev = lanes + jnp.int32(e0)
            dsp0 = jnp.zeros((16,), jnp.int32)

            def _dbody(i, carry):
                u, v, idx = carry
                for dj in range(8):
                    d = i * 8 + dj
                    a = plsc.load_gather(r0b, [ev, idx])
                    b = plsc.load_gather(r1b, [ev, idx])
                    h = a + b + dl * miscv[0, d]
                    act = jnp.maximum(h, jnp.float32(0.001) * h)
                    u = u + act * miscv[1, d]
                    v = v + act * miscv[2, d]
                    idx = idx + 1
                return (u, v, idx)

            u, v, _ = lax.fori_loop(0, D // 8, _dbody, (zer16, zer16, dsp0))
            d0 = u + miscv[3, 0]
            d1 = v + miscv[3, 1]
            dhx = drx * rs
            dhy = dry * rs
            dhz = drz * rs
            b0 = i0g * 3
            plsc.addupdate_scatter(acc, [b0], d0 * dhx)
            plsc.addupdate_scatter(acc, [b0 + 1], d0 * dhy)
            plsc.addupdate_scatter(acc, [b0 + 2], d0 * dhz)
            b1 = i1g * 3
            plsc.addupdate_scatter(acc, [b1], d1 * dhx)
            plsc.addupdate_scatter(acc, [b1 + 1], d1 * dhy)
            plsc.addupdate_scatter(acc, [b1 + 2], d1 * dhz)

    _issue(jnp.int32(0), 0)

    def _pair(p, carry):
        k0 = 2 * p
        _issue(k0 + 1, 1)
        _waitbuf(k0, 0)
        _compute(k0, 0)
        _issue(k0 + 2, 0)
        _waitbuf(k0 + 1, 1)
        _compute(k0 + 1, 1)
        return carry

    lax.fori_loop(0, (NCHUNK - 1) // 2, _pair, 0)
    _waitbuf(jnp.int32(NCHUNK - 1), 0)
    _compute(jnp.int32(NCHUNK - 1), 0)
    pltpu.sync_copy(acc, out_hbm.at[wid])


def _red_body(p_ref, a_ref, o_ref):
    o_ref[...] = a_ref[...] + jnp.sum(p_ref[...], axis=0)


_reduce = pl.pallas_call(
    _red_body,
    out_shape=jax.ShapeDtypeStruct((N3P,), jnp.float32),
)


def kernel(coords, bonds, encoded, t, answer, W0, B0, W1, B1):
    a0t = W0[:, :D].T
    a1t = W0[:, D:2 * D].T
    ch = 0.5 * (t[0] * W0[:, 2 * D] + B0)
    ch8 = jnp.zeros((8, D), jnp.float32).at[0].set(ch)
    misc = jnp.zeros((8, D), jnp.float32)
    misc = misc.at[0].set(W0[:, 2 * D + 1])
    misc = misc.at[1].set(-0.5 * W1[0])
    misc = misc.at[2].set(0.5 * W1[1])
    misc = misc.at[3, 0].set(-0.5 * B1[0])
    misc = misc.at[3, 1].set(0.5 * B1[1])

    g0, g1 = _proj(encoded, a0t, a1t, ch8)

    i0 = bonds[:, 0]
    i1 = bonds[:, 1]
    cx = coords[:, 0, 0]
    cy = coords[:, 0, 1]
    cz = coords[:, 0, 2]

    partials = _sc_edges(g0, g1, i0, i1, cx, cy, cz, misc)

    ans_pad = jnp.pad(answer.reshape(-1), (0, N3P - N3))
    out = _reduce(partials, ans_pad)
    return out[:N3].reshape(N, 1, 3)


# speedup vs baseline: 3.5853x; 1.0011x over previous
import functools

import jax
import jax.numpy as jnp
from jax import lax
from jax.experimental import pallas as pl
from jax.experimental.pallas import tpu as pltpu
from jax.experimental.pallas import tpu_sc as plsc

N = 10000
E = 320000
D = 128
NC, NS = 2, 16
NW = NC * NS
EPW = E // NW
C = 80
NCHUNK = EPW // C
GROUPS = C // 16
N3 = 3 * N
N3P = 30720


def _proj_body(x_ref, a0_ref, a1_ref, ch_ref, g0_ref, g1_ref):
    x = x_ref[...]
    ch = ch_ref[0:1, :]
    g0_ref[...] = jnp.dot(x, a0_ref[...], preferred_element_type=jnp.float32,
                        precision=lax.Precision.HIGHEST) + ch
    g1_ref[...] = jnp.dot(x, a1_ref[...], preferred_element_type=jnp.float32,
                        precision=lax.Precision.HIGHEST) + ch


_PBLK = 1000
_proj = pl.pallas_call(
    _proj_body,
    grid=(N // _PBLK,),
    in_specs=[
        pl.BlockSpec((_PBLK, D), lambda i: (i, 0)),
        pl.BlockSpec((D, D), lambda i: (0, 0)),
        pl.BlockSpec((D, D), lambda i: (0, 0)),
        pl.BlockSpec((8, D), lambda i: (0, 0)),
    ],
    out_specs=[
        pl.BlockSpec((_PBLK, D), lambda i: (i, 0)),
        pl.BlockSpec((_PBLK, D), lambda i: (i, 0)),
    ],
    out_shape=[
        jax.ShapeDtypeStruct((N, D), jnp.float32),
        jax.ShapeDtypeStruct((N, D), jnp.float32),
    ],
)


_mesh = plsc.VectorSubcoreMesh(
    core_axis_name="c", subcore_axis_name="s", num_cores=NC, num_subcores=NS)


@functools.partial(
    pl.kernel,
    out_type=jax.ShapeDtypeStruct((NW, N3P), jnp.float32),
    mesh=_mesh,
    scratch_types=[
        pltpu.VMEM((EPW,), jnp.int32),
        pltpu.VMEM((EPW,), jnp.int32),
        pltpu.VMEM((2, C, D), jnp.float32),
        pltpu.VMEM((2, C, D), jnp.float32),
        pltpu.VMEM((N,), jnp.float32),
        pltpu.VMEM((N,), jnp.float32),
        pltpu.VMEM((N,), jnp.float32),
        pltpu.VMEM((8, D), jnp.float32),
        pltpu.SMEM((8, D), jnp.float32),
        pltpu.VMEM((N3P,), jnp.float32),
        pltpu.SemaphoreType.DMA,
        pltpu.SemaphoreType.DMA,
        pltpu.SemaphoreType.DMA,
        pltpu.SemaphoreType.DMA,
        pltpu.SemaphoreType.DMA,
        pltpu.SemaphoreType.DMA,
        pltpu.SemaphoreType.DMA,
        pltpu.SemaphoreType.DMA,
    ],
    compiler_params=pltpu.CompilerParams(needs_layout_passes=False),
)
def _sc_edges(g0_hbm, g1_hbm, i0_hbm, i1_hbm, cx_hbm, cy_hbm, cz_hbm, misc_hbm,
              out_hbm, i0all, i1all, r0, r1, cxv, cyv, czv, miscb, miscv, acc,
              *sems):
    wid = lax.axis_index("s") * NC + lax.axis_index("c")
    base_w = wid * EPW

    pltpu.sync_copy(i0_hbm.at[pl.ds(base_w, EPW)], i0all)
    pltpu.sync_copy(i1_hbm.at[pl.ds(base_w, EPW)], i1all)
    pltpu.sync_copy(cx_hbm, cxv)
    pltpu.sync_copy(cy_hbm, cyv)
    pltpu.sync_copy(cz_hbm, czv)
    pltpu.sync_copy(misc_hbm, miscb)
    for i in range(3):
        for jc in range(D // 16):
            vv = miscb[i, pl.ds(jc * 16, 16)]
            for k2 in range(16):
                miscv[i, jc * 16 + k2] = vv[k2]
    vb = miscb[3, pl.ds(0, 16)]
    miscv[3, 0] = vb[0]
    miscv[3, 1] = vb[1]

    zer16 = jnp.zeros((16,), jnp.float32)

    def _zero(i, carry):
        acc[pl.ds(i * 16, 16)] = zer16
        return carry
    lax.fori_loop(0, N3P // 16, _zero, 0)

    lanes = lax.iota(jnp.int32, 16)

    H = C // 2

    def _subcopies(k, b):
        off = k * C
        yield (g0_hbm.at[i0all.at[pl.ds(off, H)]],
               r0.at[b].at[pl.ds(0, H)], sems[b])
        yield (g0_hbm.at[i0all.at[pl.ds(off + H, H)]],
               r0.at[b].at[pl.ds(H, H)], sems[2 + b])
        yield (g1_hbm.at[i1all.at[pl.ds(off, H)]],
               r1.at[b].at[pl.ds(0, H)], sems[4 + b])
        yield (g1_hbm.at[i1all.at[pl.ds(off + H, H)]],
               r1.at[b].at[pl.ds(H, H)], sems[6 + b])

    def _issue(k, b):
        for src, dst, sem in _subcopies(k, b):
            pltpu.async_copy(src, dst, sem)

    def _waitbuf(k, b):
        for src, dst, sem in _subcopies(k, b):
            pltpu.make_async_copy(src, dst, sem).wait()

    def _compute(k, b):
        r0b = r0.at[b]
        r1b = r1.at[b]
        for g in range(GROUPS):
            e0 = g * 16
            eoff = k * C + e0
            i0g = i0all[pl.ds(eoff, 16)]
            i1g = i1all[pl.ds(eoff, 16)]
            x0 = plsc.load_gather(cxv, [i0g])
            y0 = plsc.load_gather(cyv, [i0g])
            z0 = plsc.load_gather(czv, [i0g])
            x1 = plsc.load_gather(cxv, [i1g])
            y1 = plsc.load_gather(cyv, [i1g])
            z1 = plsc.load_gather(czv, [i1g])
            drx = x0 - x1
            dry = y0 - y1
            drz = z0 - z1
            dl2 = jnp.maximum(drx * drx + dry * dry + drz * drz,
                              jnp.float32(1e-12))
            xi = plsc.bitcast(dl2, jnp.int32)
            xi = jnp.int32(0x5F3759DF) - (xi >> 1)
            ys = plsc.bitcast(xi, jnp.float32)
            for _ in range(3):
                ys = ys * (jnp.float32(1.5)
                           - jnp.float32(0.5) * dl2 * ys * ys)
            rs = ys
            dl = dl2 * rs

            ev = lanes + jnp.int32(e0)
            dsp0 = jnp.zeros((16,), jnp.int32)

            def _dbody(i, carry):
                u, v, idx = carry
                for dj in range(8):
                    d = i * 8 + dj
                    a = plsc.load_gather(r0b, [ev, idx])
                    b = plsc.load_gather(r1b, [ev, idx])
                    h = a + b + dl * miscv[0, d]
                    act = jnp.maximum(h, jnp.float32(0.001) * h)
                    u = u + act * miscv[1, d]
                    v = v + act * miscv[2, d]
                    idx = idx + 1
                return (u, v, idx)

            u, v, _ = lax.fori_loop(0, D // 8, _dbody, (zer16, zer16, dsp0))
            d0 = u + miscv[3, 0]
            d1 = v + miscv[3, 1]
            dhx = drx * rs
            dhy = dry * rs
            dhz = drz * rs
            b0 = i0g * 3
            plsc.addupdate_scatter(acc, [b0], d0 * dhx)
            plsc.addupdate_scatter(acc, [b0 + 1], d0 * dhy)
            plsc.addupdate_scatter(acc, [b0 + 2], d0 * dhz)
            b1 = i1g * 3
            plsc.addupdate_scatter(acc, [b1], d1 * dhx)
            plsc.addupdate_scatter(acc, [b1 + 1], d1 * dhy)
            plsc.addupdate_scatter(acc, [b1 + 2], d1 * dhz)

    _issue(jnp.int32(0), 0)

    def _pair(p, carry):
        k0 = 2 * p
        _issue(k0 + 1, 1)
        _waitbuf(k0, 0)
        _compute(k0, 0)
        _issue(k0 + 2, 0)
        _waitbuf(k0 + 1, 1)
        _compute(k0 + 1, 1)
        return carry

    lax.fori_loop(0, (NCHUNK - 1) // 2, _pair, 0)
    _waitbuf(jnp.int32(NCHUNK - 1), 0)
    _compute(jnp.int32(NCHUNK - 1), 0)
    pltpu.sync_copy(acc, out_hbm.at[wid])


def _red_body(p_ref, a_ref, o_ref):
    o_ref[...] = a_ref[...] + jnp.sum(p_ref[...], axis=0)


_reduce = pl.pallas_call(
    _red_body,
    out_shape=jax.ShapeDtypeStruct((N3P,), jnp.float32),
)


def kernel(coords, bonds, encoded, t, answer, W0, B0, W1, B1):
    a0t = W0[:, :D].T
    a1t = W0[:, D:2 * D].T
    ch = 0.5 * (t[0] * W0[:, 2 * D] + B0)
    ch8 = jnp.zeros((8, D), jnp.float32).at[0].set(ch)
    misc = jnp.zeros((8, D), jnp.float32)
    misc = misc.at[0].set(W0[:, 2 * D + 1])
    misc = misc.at[1].set(-0.5 * W1[0])
    misc = misc.at[2].set(0.5 * W1[1])
    misc = misc.at[3, 0].set(-0.5 * B1[0])
    misc = misc.at[3, 1].set(0.5 * B1[1])

    g0, g1 = _proj(encoded, a0t, a1t, ch8)

    i0 = bonds[:, 0]
    i1 = bonds[:, 1]
    cx = coords[:, 0, 0]
    cy = coords[:, 0, 1]
    cz = coords[:, 0, 2]

    partials = _sc_edges(g0, g1, i0, i1, cx, cy, cz, misc)

    ans_pad = jnp.pad(answer.reshape(-1), (0, N3P - N3))
    out = _reduce(partials, ans_pad)
    return out[:N3].reshape(N, 1, 3)


# speedup vs baseline: 11.5936x; 3.2336x over previous
import functools

import jax
import jax.numpy as jnp
from jax import lax
from jax.experimental import pallas as pl
from jax.experimental.pallas import tpu as pltpu
from jax.experimental.pallas import tpu_sc as plsc

N = 10000
E = 320000
D = 128
NC, NS = 2, 16
NW = NC * NS
EPW = E // NW
C = 80
NCHUNK = EPW // C
GROUPS = C // 16
N3 = 3 * N
N3P = 30720


def _proj_body(x_ref, a0_ref, a1_ref, ch_ref, g0_ref, g1_ref):
    x = x_ref[...]
    ch = ch_ref[0:1, :]
    g0_ref[...] = jnp.dot(x, a0_ref[...], preferred_element_type=jnp.float32,
                        precision=lax.Precision.HIGHEST) + ch
    g1_ref[...] = jnp.dot(x, a1_ref[...], preferred_element_type=jnp.float32,
                        precision=lax.Precision.HIGHEST) + ch


_PBLK = 1000
_proj = pl.pallas_call(
    _proj_body,
    grid=(N // _PBLK,),
    in_specs=[
        pl.BlockSpec((_PBLK, D), lambda i: (i, 0)),
        pl.BlockSpec((D, D), lambda i: (0, 0)),
        pl.BlockSpec((D, D), lambda i: (0, 0)),
        pl.BlockSpec((8, D), lambda i: (0, 0)),
    ],
    out_specs=[
        pl.BlockSpec((_PBLK, D), lambda i: (i, 0)),
        pl.BlockSpec((_PBLK, D), lambda i: (i, 0)),
    ],
    out_shape=[
        jax.ShapeDtypeStruct((N, D), jnp.float32),
        jax.ShapeDtypeStruct((N, D), jnp.float32),
    ],
)


_mesh = plsc.VectorSubcoreMesh(
    core_axis_name="c", subcore_axis_name="s", num_cores=NC, num_subcores=NS)


@functools.partial(
    pl.kernel,
    out_type=jax.ShapeDtypeStruct((NW, N3P), jnp.float32),
    mesh=_mesh,
    scratch_types=[
        pltpu.VMEM((EPW,), jnp.int32),
        pltpu.VMEM((EPW,), jnp.int32),
        pltpu.VMEM((2, C, D), jnp.float32),
        pltpu.VMEM((2, C, D), jnp.float32),
        pltpu.VMEM((N,), jnp.float32),
        pltpu.VMEM((N,), jnp.float32),
        pltpu.VMEM((N,), jnp.float32),
        pltpu.VMEM((5, 2 * D), jnp.float32),
        pltpu.VMEM((N3P,), jnp.float32),
        pltpu.SemaphoreType.DMA,
        pltpu.SemaphoreType.DMA,
        pltpu.SemaphoreType.DMA,
        pltpu.SemaphoreType.DMA,
        pltpu.SemaphoreType.DMA,
        pltpu.SemaphoreType.DMA,
        pltpu.SemaphoreType.DMA,
        pltpu.SemaphoreType.DMA,
    ],
    compiler_params=pltpu.CompilerParams(needs_layout_passes=False),
)
def _sc_edges(g0_hbm, g1_hbm, i0_hbm, i1_hbm, cx_hbm, cy_hbm, cz_hbm, wext_hbm,
              out_hbm, i0all, i1all, r0, r1, cxv, cyv, czv, wextv, acc,
              *sems):
    wid = lax.axis_index("s") * NC + lax.axis_index("c")
    base_w = wid * EPW

    pltpu.sync_copy(i0_hbm.at[pl.ds(base_w, EPW)], i0all)
    pltpu.sync_copy(i1_hbm.at[pl.ds(base_w, EPW)], i1all)
    pltpu.sync_copy(cx_hbm, cxv)
    pltpu.sync_copy(cy_hbm, cyv)
    pltpu.sync_copy(cz_hbm, czv)
    pltpu.sync_copy(wext_hbm, wextv)

    zer16 = jnp.zeros((16,), jnp.float32)

    def _zero(i, carry):
        acc[pl.ds(i * 16, 16)] = zer16
        return carry
    lax.fori_loop(0, N3P // 16, _zero, 0)

    lanes = lax.iota(jnp.int32, 16)

    H = C // 2

    def _subcopies(k, b):
        off = k * C
        yield (g0_hbm.at[i0all.at[pl.ds(off, H)]],
               r0.at[b].at[pl.ds(0, H)], sems[b])
        yield (g0_hbm.at[i0all.at[pl.ds(off + H, H)]],
               r0.at[b].at[pl.ds(H, H)], sems[2 + b])
        yield (g1_hbm.at[i1all.at[pl.ds(off, H)]],
               r1.at[b].at[pl.ds(0, H)], sems[4 + b])
        yield (g1_hbm.at[i1all.at[pl.ds(off + H, H)]],
               r1.at[b].at[pl.ds(H, H)], sems[6 + b])

    def _issue(k, b):
        for src, dst, sem in _subcopies(k, b):
            pltpu.async_copy(src, dst, sem)

    def _waitbuf(k, b):
        for src, dst, sem in _subcopies(k, b):
            pltpu.make_async_copy(src, dst, sem).wait()

    def _compute(k, b):
        r0b = r0.at[b]
        r1b = r1.at[b]
        for g in range(GROUPS):
            e0 = g * 16
            eoff = k * C + e0
            i0g = i0all[pl.ds(eoff, 16)]
            i1g = i1all[pl.ds(eoff, 16)]
            x0 = plsc.load_gather(cxv, [i0g])
            y0 = plsc.load_gather(cyv, [i0g])
            z0 = plsc.load_gather(czv, [i0g])
            x1 = plsc.load_gather(cxv, [i1g])
            y1 = plsc.load_gather(cyv, [i1g])
            z1 = plsc.load_gather(czv, [i1g])
            drx = x0 - x1
            dry = y0 - y1
            drz = z0 - z1
            dl2 = jnp.maximum(drx * drx + dry * dry + drz * drz,
                              jnp.float32(1e-12))
            xi = plsc.bitcast(dl2, jnp.int32)
            xi = jnp.int32(0x5F3759DF) - (xi >> 1)
            ys = plsc.bitcast(xi, jnp.float32)
            for _ in range(3):
                ys = ys * (jnp.float32(1.5)
                           - jnp.float32(0.5) * dl2 * ys * ys)
            rs = ys
            dl = dl2 * rs

            ev = lanes + jnp.int32(e0)

            def _bbody(bb, uv):
                u, v = uv
                bb16 = bb * 16
                bb32 = bb * 32
                for s in range(16):
                    dcol = ((lanes + jnp.int32(s)) & jnp.int32(15)) + bb16
                    a = plsc.load_gather(r0b, [ev, dcol])
                    b = plsc.load_gather(r1b, [ev, dcol])
                    wdlv = wextv[0, pl.ds(bb32 + s, 16)]
                    w0v = wextv[1, pl.ds(bb32 + s, 16)]
                    w1v = wextv[2, pl.ds(bb32 + s, 16)]
                    h = a + b + dl * wdlv
                    act = jnp.maximum(h, jnp.float32(0.001) * h)
                    u = u + act * w0v
                    v = v + act * w1v
                return (u, v)

            u, v = lax.fori_loop(0, D // 16, _bbody, (zer16, zer16))
            d0 = u + wextv[3, pl.ds(0, 16)]
            d1 = v + wextv[4, pl.ds(0, 16)]
            dhx = drx * rs
            dhy = dry * rs
            dhz = drz * rs
            b0 = i0g * 3
            plsc.addupdate_scatter(acc, [b0], d0 * dhx)
            plsc.addupdate_scatter(acc, [b0 + 1], d0 * dhy)
            plsc.addupdate_scatter(acc, [b0 + 2], d0 * dhz)
            b1 = i1g * 3
            plsc.addupdate_scatter(acc, [b1], d1 * dhx)
            plsc.addupdate_scatter(acc, [b1 + 1], d1 * dhy)
            plsc.addupdate_scatter(acc, [b1 + 2], d1 * dhz)

    _issue(jnp.int32(0), 0)

    def _pair(p, carry):
        k0 = 2 * p
        _issue(k0 + 1, 1)
        _waitbuf(k0, 0)
        _compute(k0, 0)
        _issue(k0 + 2, 0)
        _waitbuf(k0 + 1, 1)
        _compute(k0 + 1, 1)
        return carry

    lax.fori_loop(0, (NCHUNK - 1) // 2, _pair, 0)
    _waitbuf(jnp.int32(NCHUNK - 1), 0)
    _compute(jnp.int32(NCHUNK - 1), 0)
    pltpu.sync_copy(acc, out_hbm.at[wid])


def _red_body(p_ref, a_ref, o_ref):
    o_ref[...] = a_ref[...] + jnp.sum(p_ref[...], axis=0)


_reduce = pl.pallas_call(
    _red_body,
    out_shape=jax.ShapeDtypeStruct((N3P,), jnp.float32),
)


def kernel(coords, bonds, encoded, t, answer, W0, B0, W1, B1):
    a0t = W0[:, :D].T
    a1t = W0[:, D:2 * D].T
    ch = 0.5 * (t[0] * W0[:, 2 * D] + B0)
    ch8 = jnp.zeros((8, D), jnp.float32).at[0].set(ch)

    def _ext(w):
        return jnp.tile(w.reshape(8, 16), (1, 2)).reshape(-1)

    wext = jnp.stack([
        _ext(W0[:, 2 * D + 1]),
        _ext(-0.5 * W1[0]),
        _ext(0.5 * W1[1]),
        jnp.full((2 * D,), -0.5 * B1[0], jnp.float32),
        jnp.full((2 * D,), 0.5 * B1[1], jnp.float32),
    ])

    g0, g1 = _proj(encoded, a0t, a1t, ch8)

    i0 = bonds[:, 0]
    i1 = bonds[:, 1]
    cx = coords[:, 0, 0]
    cy = coords[:, 0, 1]
    cz = coords[:, 0, 2]

    partials = _sc_edges(g0, g1, i0, i1, cx, cy, cz, wext)

    ans_pad = jnp.pad(answer.reshape(-1), (0, N3P - N3))
    out = _reduce(partials, ans_pad)
    return out[:N3].reshape(N, 1, 3)
